# baseline (speedup 1.0000x reference)
import functools
import jax
import jax.numpy as jnp
from jax import lax
from jax.experimental import pallas as pl
from jax.experimental.pallas import tpu as pltpu, tpu_sc as plsc

N = 100000
E = 1600000
NPAD = 100352
NT = 16
EPT = E // NT
SUB = 80
NSUB = 10
CH = SUB * NSUB
NCHUNK = EPT // CH
ROWS2D = E // SUB
STRIPE = N // NT
ZPIECE = 625
DSTRIPE = NPAD // NT
K_CHEB = 7
FA = 64

_MESH = plsc.VectorSubcoreMesh(core_axis_name="c", subcore_axis_name="s")
_SC_PARAMS = pltpu.CompilerParams(use_tc_tiling_on_sc=False,
                                  needs_layout_passes=False)


def _deg_body(src2, dst2, ew2, out_h, sbuf, dbuf, vbuf, lacc):
    ci = lax.axis_index("c")
    w = lax.axis_index("s")
    cif = (1 - ci).astype(jnp.float32)

    def zloop(j, c):
        lacc[pl.ds(j * 16, 16)] = jnp.zeros((16,), jnp.float32)
        return c
    lax.fori_loop(0, NPAD // 16, zloop, 0)

    def chunk(t, c):
        rb = w * (EPT // SUB) + t * NSUB
        pltpu.sync_copy(src2.at[pl.ds(rb, NSUB)], sbuf)
        pltpu.sync_copy(dst2.at[pl.ds(rb, NSUB)], dbuf)
        pltpu.sync_copy(ew2.at[pl.ds(rb, NSUB)], vbuf)
        for i in range(NSUB):
            for j in range(SUB // 16):
                s16 = sbuf[i, pl.ds(j * 16, 16)]
                d16 = dbuf[i, pl.ds(j * 16, 16)]
                e16 = vbuf[i, pl.ds(j * 16, 16)]
                wm = jnp.where(s16 == d16, 0.0, e16)
                sel = s16 * (1 - ci) + d16 * ci
                val = wm * cif + (1.0 - cif)
                plsc.addupdate_scatter(lacc, [sel], val)
        return c
    lax.fori_loop(0, NCHUNK, chunk, 0)
    pltpu.sync_copy(lacc, out_h.at[(ci * NT + w)])


_deg_call = pl.kernel(
    _deg_body,
    out_type=jax.ShapeDtypeStruct((2 * NT, NPAD), jnp.float32),
    mesh=_MESH,
    compiler_params=_SC_PARAMS,
    scratch_types=[
        pltpu.VMEM((NSUB, SUB), jnp.int32),
        pltpu.VMEM((NSUB, SUB), jnp.int32),
        pltpu.VMEM((NSUB, SUB), jnp.float32),
        pltpu.VMEM((NPAD,), jnp.float32),
    ],
)


def _norm_body(src2, dst2, ew2, dinv2_h, out_h, sbuf, dbuf, ebuf, obuf, g1, g2,
               sem):
    ci = lax.axis_index("c")
    w = lax.axis_index("s")
    cif = (1 - ci).astype(jnp.float32)
    sign = 1.0 - 2.0 * cif
    ciNP = ci * NPAD

    def chunk(t, c):
        rb = w * (EPT // SUB) + t * NSUB
        pltpu.sync_copy(src2.at[pl.ds(rb, NSUB)], sbuf)
        pltpu.sync_copy(dst2.at[pl.ds(rb, NSUB)], dbuf)
        pltpu.sync_copy(ew2.at[pl.ds(rb, NSUB)], ebuf)
        for i in range(NSUB):
            for j in range(SUB // 16):
                sbuf[i, pl.ds(j * 16, 16)] = sbuf[i, pl.ds(j * 16, 16)] + ciNP
                dbuf[i, pl.ds(j * 16, 16)] = dbuf[i, pl.ds(j * 16, 16)] + ciNP
        descs = []
        for i in range(NSUB):
            descs.append(pltpu.async_copy(dinv2_h.at[sbuf.at[i]], g1.at[i], sem))
            descs.append(pltpu.async_copy(dinv2_h.at[dbuf.at[i]], g2.at[i], sem))
        for d in descs:
            d.wait()
        for i in range(NSUB):
            for j in range(SUB // 16):
                s16 = sbuf[i, pl.ds(j * 16, 16)]
                d16 = dbuf[i, pl.ds(j * 16, 16)]
                e16 = ebuf[i, pl.ds(j * 16, 16)]
                d1 = g1[i, pl.ds(j * 16, 16)]
                d2 = g2[i, pl.ds(j * 16, 16)]
                wm = jnp.where(s16 == d16, 0.0, e16)
                weff = wm * cif + (1.0 - cif)
                obuf[i, pl.ds(j * 16, 16)] = sign * (d1 * d2 * weff)
        pltpu.sync_copy(obuf, out_h.at[pl.ds(ci * ROWS2D + rb, NSUB)])
        return c
    lax.fori_loop(0, NCHUNK, chunk, 0)


_norm_call = pl.kernel(
    _norm_body,
    out_type=jax.ShapeDtypeStruct((2 * ROWS2D, SUB), jnp.float32),
    mesh=_MESH,
    compiler_params=_SC_PARAMS,
    scratch_types=[
        pltpu.VMEM((NSUB, SUB), jnp.int32),
        pltpu.VMEM((NSUB, SUB), jnp.int32),
        pltpu.VMEM((NSUB, SUB), jnp.float32),
        pltpu.VMEM((NSUB, SUB), jnp.float32),
        pltpu.VMEM((NSUB, SUB), jnp.float32),
        pltpu.VMEM((NSUB, SUB), jnp.float32),
        pltpu.SemaphoreType.DMA,
    ],
)


def _prop_body(z_h, src2, dst2, wn2, zeros_h, out_h, sbuf, dbuf, wb, rows,
               acc, sem0, sem1):
    ci = lax.axis_index("c")
    w = lax.axis_index("s")
    ciN = ci * N
    sems = (sem0, sem1)

    pltpu.sync_copy(zeros_h.at[pl.ds(w * STRIPE, STRIPE)],
                    acc.at[pl.ds(w * STRIPE, STRIPE)])
    plsc.subcore_barrier()

    def load_fire(t, b):
        rb = w * (EPT // SUB) + t * NSUB
        pltpu.sync_copy(src2.at[pl.ds(rb, NSUB)], sbuf.at[b])
        pltpu.sync_copy(dst2.at[pl.ds(rb, NSUB)], dbuf.at[b])
        pltpu.sync_copy(wn2.at[pl.ds(rb, NSUB)], wb.at[b])
        for i in range(NSUB):
            for j in range(SUB // 16):
                s16 = sbuf[b, i, pl.ds(j * 16, 16)]
                sbuf[b, i, pl.ds(j * 16, 16)] = s16 + ciN
        for i in range(NSUB):
            pltpu.async_copy(z_h.at[sbuf.at[b, i]],
                             rows.at[b, pl.ds(i * SUB, SUB)], sems[b])

    def drain(b):
        for i in range(NSUB):
            pltpu.make_async_copy(z_h.at[sbuf.at[b, i]],
                                  rows.at[b, pl.ds(i * SUB, SUB)],
                                  sems[b]).wait()

    def consume(b):
        def scale_i(i, c2):
            def scale_g(g, c3):
                base = i * SUB + g * 16
                w16 = wb[b, i, pl.ds(g * 16, 16)]
                for l in range(16):
                    rows[b, base + l, :] = rows[b, base + l, :] * w16[l]
                return c3
            return lax.fori_loop(0, SUB // 16, scale_g, c2)
        lax.fori_loop(0, NSUB, scale_i, 0)
        for i in range(NSUB):
            pltpu.sync_copy(rows.at[b, pl.ds(i * SUB, SUB)],
                            acc.at[dbuf.at[b, i]], add=True)

    load_fire(0, 0)

    def chunk2(u, c):
        t = 2 * u
        load_fire(t + 1, 1)
        drain(0)
        consume(0)
        load_fire(t + 2, 0)
        drain(1)
        consume(1)
        return c
    lax.fori_loop(0, (NCHUNK - 1) // 2, chunk2, 0)
    drain(0)
    consume(0)
    plsc.subcore_barrier()
    pltpu.sync_copy(acc.at[pl.ds(w * STRIPE, STRIPE)],
                    out_h.at[pl.ds(ciN + w * STRIPE, STRIPE)])


_prop_call = pl.kernel(
    _prop_body,
    out_type=jax.ShapeDtypeStruct((2 * N, 16), jnp.float32),
    mesh=_MESH,
    compiler_params=_SC_PARAMS,
    scratch_types=[
        pltpu.VMEM((2, NSUB, SUB), jnp.int32),
        pltpu.VMEM((2, NSUB, SUB), jnp.int32),
        pltpu.VMEM((2, NSUB, SUB), jnp.float32),
        pltpu.VMEM((2, CH, 16), jnp.float32),
        pltpu.VMEM_SHARED((N, 16), jnp.float32),
        pltpu.SemaphoreType.DMA,
        pltpu.SemaphoreType.DMA,
    ],
)


_BLK = 2000


def _tc(body, out_shapes, in_specs, out_specs, grid):
    return pl.pallas_call(body, grid=grid, in_specs=in_specs,
                          out_specs=out_specs, out_shape=out_shapes)


def _rep(shape):
    return pl.BlockSpec(shape, lambda i: tuple(0 for _ in shape))


def _first_body(x_ref, wf_ref, bf_ref, h2_ref):
    h = jnp.maximum(x_ref[...] @ wf_ref[...] + bf_ref[...], 0.0)
    h2_ref[0] = h[:, :16]
    h2_ref[1] = h[:, 16:]


def _dinv_body(deg_ref, out_ref):
    d = jnp.sum(deg_ref[...], axis=1)
    safe = jnp.where(d > 0, d, 1.0)
    out_ref[...] = jnp.where(d > 0, lax.rsqrt(safe), 0.0)


def _cheb1_body(h2_ref, p2_ref, w0_ref, w1_ref, oc_ref):
    h = jnp.concatenate([h2_ref[0], h2_ref[1]], axis=1)
    p = jnp.concatenate([p2_ref[0], p2_ref[1]], axis=1)
    oc_ref[...] = h @ w0_ref[...] + p @ w1_ref[...]


def _chebk_body(last, p2_ref, tx0_ref, wk_ref, ocin_ref, cb_ref, tx2_ref, oc_ref):
    tx2 = 2.0 * p2_ref[...] - tx0_ref[...]
    tx2_ref[...] = tx2
    xcat = jnp.concatenate([tx2[0], tx2[1]], axis=1)
    oc = ocin_ref[...] + xcat @ wk_ref[...]
    if last:
        oc = jnp.maximum(oc + cb_ref[...], 0.0)
    oc_ref[...] = oc


def _apre_body(h2_ref, ip_ref, z8_ref):
    h = jnp.concatenate([h2_ref[0], h2_ref[1]], axis=1)
    for s in range(2):
        y = h @ ip_ref[s]
        for c in range(4):
            z8_ref[s * 4 + c] = y[:, c * 16:(c + 1) * 16]


def _amid_body(p8_ref, h2_ref, r0_ref, b0_ref, w0_ref, z8_ref):
    h = jnp.concatenate([h2_ref[0], h2_ref[1]], axis=1)
    for s in range(2):
        p = jnp.concatenate([p8_ref[s * 4 + c] for c in range(4)], axis=1)
        t = jnp.maximum(p + h @ r0_ref[s] + b0_ref[s], 0.0)
        y = t @ w0_ref[s]
        for c in range(4):
            z8_ref[s * 4 + c] = y[:, c * 16:(c + 1) * 16]


def _afin_body(p8_ref, h2_ref, r1_ref, b1_ref, x1_ref, wo_ref, bo_ref, o_ref):
    h = jnp.concatenate([h2_ref[0], h2_ref[1]], axis=1)
    ts = []
    for s in range(2):
        p = jnp.concatenate([p8_ref[s * 4 + c] for c in range(4)], axis=1)
        ts.append(jnp.maximum(p + h @ r1_ref[s] + b1_ref[s], 0.0))
    x2 = jnp.maximum((ts[0] + ts[1]) * 0.5, 0.0)[:, :50]
    xc = jnp.concatenate([x1_ref[...], x2], axis=1)
    o_ref[...] = xc @ wo_ref[...] + bo_ref[...]


def _pad(a, axes_pads):
    return jnp.pad(a, axes_pads)


def kernel(x, edge_index, edge_weight, W_first, b_first, cheb_W, cheb_b,
           arma_init, arma_W, arma_root, arma_b, W_out, b_out):
    src2 = edge_index[0].reshape(ROWS2D, SUB)
    dst2 = edge_index[1].reshape(ROWS2D, SUB)

    ew2 = edge_weight.reshape(ROWS2D, SUB)
    deg2 = _deg_call(src2, dst2, ew2)
    dinv2 = _tc(
        _dinv_body,
        jax.ShapeDtypeStruct((2, NPAD), jnp.float32),
        [pl.BlockSpec((2, NT, DSTRIPE), lambda i: (0, 0, i))],
        pl.BlockSpec((2, DSTRIPE), lambda i: (0, i)),
        (NPAD // DSTRIPE,),
    )(deg2.reshape(2, NT, NPAD)).reshape(2 * NPAD)
    norm2 = _norm_call(src2, dst2, ew2, dinv2)
    norm = norm2[:ROWS2D]
    norma = norm2[ROWS2D:]

    g = (N // _BLK,)
    bspec2 = pl.BlockSpec((2, _BLK, 16), lambda i: (0, i, 0))
    bspec8 = pl.BlockSpec((8, _BLK, 16), lambda i: (0, i, 0))
    h2 = _tc(
        _first_body,
        jax.ShapeDtypeStruct((2, N, 16), jnp.float32),
        [pl.BlockSpec((_BLK, 32), lambda i: (i, 0)), _rep((32, 32)), _rep((1, 32))],
        bspec2, g,
    )(x, W_first, b_first.reshape(1, 32))
    h2f = h2.reshape(2 * N, 16)

    zer = jnp.zeros((N, 16), jnp.float32)
    P = _prop_call(h2f, src2, dst2, norm, zer)
    oc = _tc(
        _cheb1_body,
        jax.ShapeDtypeStruct((N, 50), jnp.float32),
        [bspec2, bspec2, _rep((32, 50)), _rep((32, 50))],
        pl.BlockSpec((_BLK, 50), lambda i: (i, 0)), g,
    )(h2, P.reshape(2, N, 16), cheb_W[0], cheb_W[1])
    tx0, tx1 = h2f, P
    for k in range(2, K_CHEB):
        Pk = _prop_call(tx1, src2, dst2, norm, zer)
        tx2, oc = _tc(
            functools.partial(_chebk_body, k == K_CHEB - 1),
            (jax.ShapeDtypeStruct((2, N, 16), jnp.float32),
             jax.ShapeDtypeStruct((N, 50), jnp.float32)),
            [bspec2, bspec2, _rep((32, 50)),
             pl.BlockSpec((_BLK, 50), lambda i: (i, 0)), _rep((1, 50))],
            (bspec2, pl.BlockSpec((_BLK, 50), lambda i: (i, 0))), g,
        )(Pk.reshape(2, N, 16), tx0.reshape(2, N, 16), cheb_W[k], oc,
          cheb_b.reshape(1, 50))
        tx0, tx1 = tx1, tx2.reshape(2 * N, 16)
    x1 = oc

    ip = _pad(arma_init, ((0, 0), (0, 0), (0, 14)))
    w0p = _pad(arma_W[0], ((0, 0), (0, 14), (0, 14)))
    r0p = _pad(arma_root[0], ((0, 0), (0, 0), (0, 14)))
    r1p = _pad(arma_root[1], ((0, 0), (0, 0), (0, 14)))
    b0p = _pad(arma_b[0].reshape(2, 50), ((0, 0), (0, 14)))
    b1p = _pad(arma_b[1].reshape(2, 50), ((0, 0), (0, 14)))

    z8 = _tc(
        _apre_body,
        jax.ShapeDtypeStruct((8, N, 16), jnp.float32),
        [bspec2, _rep((2, 32, 64))],
        bspec8, g,
    )(h2, ip).reshape(8 * N, 16)
    P8 = jnp.concatenate(
        [_prop_call(z8[2 * N * p:2 * N * (p + 1)], src2, dst2, norma, zer)
         for p in range(4)], axis=0)
    z8b = _tc(
        _amid_body,
        jax.ShapeDtypeStruct((8, N, 16), jnp.float32),
        [bspec8, bspec2, _rep((2, 32, 64)), _rep((2, 64)), _rep((2, 64, 64))],
        bspec8, g,
    )(P8.reshape(8, N, 16), h2, r0p, b0p, w0p).reshape(8 * N, 16)
    P8b = jnp.concatenate(
        [_prop_call(z8b[2 * N * p:2 * N * (p + 1)], src2, dst2, norma, zer)
         for p in range(4)], axis=0)
    out = _tc(
        _afin_body,
        jax.ShapeDtypeStruct((N, 2), jnp.float32),
        [bspec8, bspec2, _rep((2, 32, 64)), _rep((2, 64)),
         pl.BlockSpec((_BLK, 50), lambda i: (i, 0)), _rep((100, 2)), _rep((1, 2))],
        pl.BlockSpec((_BLK, 2), lambda i: (i, 0)), g,
    )(P8b.reshape(8, N, 16), h2, r1p, b1p, x1, W_out, b_out.reshape(1, 2))
    return out


# speedup vs baseline: 1.0296x; 1.0296x over previous
import functools
import jax
import jax.numpy as jnp
from jax import lax
from jax.experimental import pallas as pl
from jax.experimental.pallas import tpu as pltpu, tpu_sc as plsc

N = 100000
E = 1600000
NPAD = 100352
NT = 16
EPT = E // NT
SUB = 80
NSUB = 10
CH = SUB * NSUB
NCHUNK = EPT // CH
ROWS2D = E // SUB
STRIPE = N // NT
ZPIECE = 625
DSTRIPE = NPAD // NT
K_CHEB = 7
FA = 64

_MESH = plsc.VectorSubcoreMesh(core_axis_name="c", subcore_axis_name="s")
_SC_PARAMS = pltpu.CompilerParams(use_tc_tiling_on_sc=False,
                                  needs_layout_passes=False)


def _deg_body(src2, dst2, ew2, out_h, sbuf, dbuf, selb, vbuf, zb, acc):
    ci = lax.axis_index("c")
    w = lax.axis_index("s")
    cif = (1 - ci).astype(jnp.float32)

    def zloop(j, c):
        zb[pl.ds(j * 16, 16)] = jnp.zeros((16,), jnp.float32)
        return c
    lax.fori_loop(0, DSTRIPE // 16, zloop, 0)
    pltpu.sync_copy(zb, acc.at[pl.ds(w * DSTRIPE, DSTRIPE)])
    plsc.subcore_barrier()

    def chunk(t, c):
        rb = w * (EPT // SUB) + t * NSUB
        pltpu.sync_copy(src2.at[pl.ds(rb, NSUB)], sbuf)
        pltpu.sync_copy(dst2.at[pl.ds(rb, NSUB)], dbuf)
        pltpu.sync_copy(ew2.at[pl.ds(rb, NSUB)], vbuf)
        for i in range(NSUB):
            for j in range(SUB // 16):
                s16 = sbuf[i, pl.ds(j * 16, 16)]
                d16 = dbuf[i, pl.ds(j * 16, 16)]
                e16 = vbuf[i, pl.ds(j * 16, 16)]
                wm = jnp.where(s16 == d16, 0.0, e16)
                selb[i, pl.ds(j * 16, 16)] = s16 * (1 - ci) + d16 * ci
                vbuf[i, pl.ds(j * 16, 16)] = wm * cif + (1.0 - cif)
        for i in range(NSUB):
            pltpu.sync_copy(vbuf.at[i], acc.at[selb.at[i]], add=True)
        return c
    lax.fori_loop(0, NCHUNK, chunk, 0)
    plsc.subcore_barrier()
    pltpu.sync_copy(acc.at[pl.ds(w * DSTRIPE, DSTRIPE)], zb)
    pltpu.sync_copy(zb, out_h.at[pl.ds(ci * NPAD + w * DSTRIPE, DSTRIPE)])


_deg_call = pl.kernel(
    _deg_body,
    out_type=jax.ShapeDtypeStruct((2 * NPAD,), jnp.float32),
    mesh=_MESH,
    compiler_params=_SC_PARAMS,
    scratch_types=[
        pltpu.VMEM((NSUB, SUB), jnp.int32),
        pltpu.VMEM((NSUB, SUB), jnp.int32),
        pltpu.VMEM((NSUB, SUB), jnp.int32),
        pltpu.VMEM((NSUB, SUB), jnp.float32),
        pltpu.VMEM((DSTRIPE,), jnp.float32),
        pltpu.VMEM_SHARED((NPAD,), jnp.float32),
    ],
)


def _norm_body(src2, dst2, ew2, dinv2_h, out_h, sbuf, dbuf, ebuf, obuf, dl):
    ci = lax.axis_index("c")
    w = lax.axis_index("s")
    cif = (1 - ci).astype(jnp.float32)
    sign = 1.0 - 2.0 * cif
    pltpu.sync_copy(dinv2_h.at[pl.ds(ci * NPAD, NPAD)], dl)

    def chunk(t, c):
        rb = w * (EPT // SUB) + t * NSUB
        pltpu.sync_copy(src2.at[pl.ds(rb, NSUB)], sbuf)
        pltpu.sync_copy(dst2.at[pl.ds(rb, NSUB)], dbuf)
        pltpu.sync_copy(ew2.at[pl.ds(rb, NSUB)], ebuf)
        for i in range(NSUB):
            for j in range(SUB // 16):
                s16 = sbuf[i, pl.ds(j * 16, 16)]
                d16 = dbuf[i, pl.ds(j * 16, 16)]
                e16 = ebuf[i, pl.ds(j * 16, 16)]
                d1 = plsc.load_gather(dl, [s16])
                d2 = plsc.load_gather(dl, [d16])
                wm = jnp.where(s16 == d16, 0.0, e16)
                weff = wm * cif + (1.0 - cif)
                obuf[i, pl.ds(j * 16, 16)] = sign * (d1 * d2 * weff)
        pltpu.sync_copy(obuf, out_h.at[pl.ds(ci * ROWS2D + rb, NSUB)])
        return c
    lax.fori_loop(0, NCHUNK, chunk, 0)


_norm_call = pl.kernel(
    _norm_body,
    out_type=jax.ShapeDtypeStruct((2 * ROWS2D, SUB), jnp.float32),
    mesh=_MESH,
    compiler_params=_SC_PARAMS,
    scratch_types=[
        pltpu.VMEM((NSUB, SUB), jnp.int32),
        pltpu.VMEM((NSUB, SUB), jnp.int32),
        pltpu.VMEM((NSUB, SUB), jnp.float32),
        pltpu.VMEM((NSUB, SUB), jnp.float32),
        pltpu.VMEM((NPAD,), jnp.float32),
    ],
)


def _prop_body(z_h, src2, dst2, wn2, zeros_h, out_h, sbuf, dbuf, wb, rows,
               acc, sem0, sem1):
    ci = lax.axis_index("c")
    w = lax.axis_index("s")
    ciN = ci * N
    sems = (sem0, sem1)

    pltpu.sync_copy(zeros_h.at[pl.ds(w * STRIPE, STRIPE)],
                    acc.at[pl.ds(w * STRIPE, STRIPE)])
    plsc.subcore_barrier()

    def load_fire(t, b):
        rb = w * (EPT // SUB) + t * NSUB
        pltpu.sync_copy(src2.at[pl.ds(rb, NSUB)], sbuf.at[b])
        pltpu.sync_copy(dst2.at[pl.ds(rb, NSUB)], dbuf.at[b])
        pltpu.sync_copy(wn2.at[pl.ds(rb, NSUB)], wb.at[b])
        for i in range(NSUB):
            for j in range(SUB // 16):
                s16 = sbuf[b, i, pl.ds(j * 16, 16)]
                sbuf[b, i, pl.ds(j * 16, 16)] = s16 + ciN
        for i in range(NSUB):
            pltpu.async_copy(z_h.at[sbuf.at[b, i]],
                             rows.at[b, pl.ds(i * SUB, SUB)], sems[b])

    def drain(b):
        for i in range(NSUB):
            pltpu.make_async_copy(z_h.at[sbuf.at[b, i]],
                                  rows.at[b, pl.ds(i * SUB, SUB)],
                                  sems[b]).wait()

    def consume(b):
        def scale_i(i, c2):
            def scale_g(g, c3):
                base = i * SUB + g * 16
                w16 = wb[b, i, pl.ds(g * 16, 16)]
                for l in range(16):
                    rows[b, base + l, :] = rows[b, base + l, :] * w16[l]
                return c3
            return lax.fori_loop(0, SUB // 16, scale_g, c2)
        lax.fori_loop(0, NSUB, scale_i, 0)
        for i in range(NSUB):
            pltpu.sync_copy(rows.at[b, pl.ds(i * SUB, SUB)],
                            acc.at[dbuf.at[b, i]], add=True)

    load_fire(0, 0)

    def chunk2(u, c):
        t = 2 * u
        load_fire(t + 1, 1)
        drain(0)
        consume(0)
        load_fire(t + 2, 0)
        drain(1)
        consume(1)
        return c
    lax.fori_loop(0, (NCHUNK - 1) // 2, chunk2, 0)
    drain(0)
    consume(0)
    plsc.subcore_barrier()
    pltpu.sync_copy(acc.at[pl.ds(w * STRIPE, STRIPE)],
                    out_h.at[pl.ds(ciN + w * STRIPE, STRIPE)])


_prop_call = pl.kernel(
    _prop_body,
    out_type=jax.ShapeDtypeStruct((2 * N, 16), jnp.float32),
    mesh=_MESH,
    compiler_params=_SC_PARAMS,
    scratch_types=[
        pltpu.VMEM((2, NSUB, SUB), jnp.int32),
        pltpu.VMEM((2, NSUB, SUB), jnp.int32),
        pltpu.VMEM((2, NSUB, SUB), jnp.float32),
        pltpu.VMEM((2, CH, 16), jnp.float32),
        pltpu.VMEM_SHARED((N, 16), jnp.float32),
        pltpu.SemaphoreType.DMA,
        pltpu.SemaphoreType.DMA,
    ],
)


_BLK = 2000


def _tc(body, out_shapes, in_specs, out_specs, grid):
    return pl.pallas_call(body, grid=grid, in_specs=in_specs,
                          out_specs=out_specs, out_shape=out_shapes)


def _rep(shape):
    return pl.BlockSpec(shape, lambda i: tuple(0 for _ in shape))


def _first_body(x_ref, wf_ref, bf_ref, h2_ref):
    h = jnp.maximum(x_ref[...] @ wf_ref[...] + bf_ref[...], 0.0)
    h2_ref[0] = h[:, :16]
    h2_ref[1] = h[:, 16:]


def _dinv_body(deg_ref, out_ref):
    d = deg_ref[...]
    safe = jnp.where(d > 0, d, 1.0)
    out_ref[...] = jnp.where(d > 0, lax.rsqrt(safe), 0.0)


def _cheb1_body(h2_ref, p2_ref, w0_ref, w1_ref, oc_ref):
    h = jnp.concatenate([h2_ref[0], h2_ref[1]], axis=1)
    p = jnp.concatenate([p2_ref[0], p2_ref[1]], axis=1)
    oc_ref[...] = h @ w0_ref[...] + p @ w1_ref[...]


def _chebk_body(last, p2_ref, tx0_ref, wk_ref, ocin_ref, cb_ref, tx2_ref, oc_ref):
    tx2 = 2.0 * p2_ref[...] - tx0_ref[...]
    tx2_ref[...] = tx2
    xcat = jnp.concatenate([tx2[0], tx2[1]], axis=1)
    oc = ocin_ref[...] + xcat @ wk_ref[...]
    if last:
        oc = jnp.maximum(oc + cb_ref[...], 0.0)
    oc_ref[...] = oc


def _apre_body(h2_ref, ip_ref, z8_ref):
    h = jnp.concatenate([h2_ref[0], h2_ref[1]], axis=1)
    for s in range(2):
        y = h @ ip_ref[s]
        for c in range(4):
            z8_ref[s * 4 + c] = y[:, c * 16:(c + 1) * 16]


def _amid_body(p8_ref, h2_ref, r0_ref, b0_ref, w0_ref, z8_ref):
    h = jnp.concatenate([h2_ref[0], h2_ref[1]], axis=1)
    for s in range(2):
        p = jnp.concatenate([p8_ref[s * 4 + c] for c in range(4)], axis=1)
        t = jnp.maximum(p + h @ r0_ref[s] + b0_ref[s], 0.0)
        y = t @ w0_ref[s]
        for c in range(4):
            z8_ref[s * 4 + c] = y[:, c * 16:(c + 1) * 16]


def _afin_body(p8_ref, h2_ref, r1_ref, b1_ref, x1_ref, wo_ref, bo_ref, o_ref):
    h = jnp.concatenate([h2_ref[0], h2_ref[1]], axis=1)
    ts = []
    for s in range(2):
        p = jnp.concatenate([p8_ref[s * 4 + c] for c in range(4)], axis=1)
        ts.append(jnp.maximum(p + h @ r1_ref[s] + b1_ref[s], 0.0))
    x2 = jnp.maximum((ts[0] + ts[1]) * 0.5, 0.0)[:, :50]
    xc = jnp.concatenate([x1_ref[...], x2], axis=1)
    o_ref[...] = xc @ wo_ref[...] + bo_ref[...]


def _pad(a, axes_pads):
    return jnp.pad(a, axes_pads)


def kernel(x, edge_index, edge_weight, W_first, b_first, cheb_W, cheb_b,
           arma_init, arma_W, arma_root, arma_b, W_out, b_out):
    src2 = edge_index[0].reshape(ROWS2D, SUB)
    dst2 = edge_index[1].reshape(ROWS2D, SUB)

    ew2 = edge_weight.reshape(ROWS2D, SUB)
    deg2 = _deg_call(src2, dst2, ew2)
    dinv2 = _tc(
        _dinv_body,
        jax.ShapeDtypeStruct((2, NPAD), jnp.float32),
        [pl.BlockSpec((2, DSTRIPE), lambda i: (0, i))],
        pl.BlockSpec((2, DSTRIPE), lambda i: (0, i)),
        (NPAD // DSTRIPE,),
    )(deg2.reshape(2, NPAD)).reshape(2 * NPAD)
    norm2 = _norm_call(src2, dst2, ew2, dinv2)
    norm = norm2[:ROWS2D]
    norma = norm2[ROWS2D:]

    g = (N // _BLK,)
    bspec2 = pl.BlockSpec((2, _BLK, 16), lambda i: (0, i, 0))
    bspec8 = pl.BlockSpec((8, _BLK, 16), lambda i: (0, i, 0))
    h2 = _tc(
        _first_body,
        jax.ShapeDtypeStruct((2, N, 16), jnp.float32),
        [pl.BlockSpec((_BLK, 32), lambda i: (i, 0)), _rep((32, 32)), _rep((1, 32))],
        bspec2, g,
    )(x, W_first, b_first.reshape(1, 32))
    h2f = h2.reshape(2 * N, 16)

    zer = jnp.zeros((N, 16), jnp.float32)
    P = _prop_call(h2f, src2, dst2, norm, zer)
    oc = _tc(
        _cheb1_body,
        jax.ShapeDtypeStruct((N, 50), jnp.float32),
        [bspec2, bspec2, _rep((32, 50)), _rep((32, 50))],
        pl.BlockSpec((_BLK, 50), lambda i: (i, 0)), g,
    )(h2, P.reshape(2, N, 16), cheb_W[0], cheb_W[1])
    tx0, tx1 = h2f, P
    for k in range(2, K_CHEB):
        Pk = _prop_call(tx1, src2, dst2, norm, zer)
        tx2, oc = _tc(
            functools.partial(_chebk_body, k == K_CHEB - 1),
            (jax.ShapeDtypeStruct((2, N, 16), jnp.float32),
             jax.ShapeDtypeStruct((N, 50), jnp.float32)),
            [bspec2, bspec2, _rep((32, 50)),
             pl.BlockSpec((_BLK, 50), lambda i: (i, 0)), _rep((1, 50))],
            (bspec2, pl.BlockSpec((_BLK, 50), lambda i: (i, 0))), g,
        )(Pk.reshape(2, N, 16), tx0.reshape(2, N, 16), cheb_W[k], oc,
          cheb_b.reshape(1, 50))
        tx0, tx1 = tx1, tx2.reshape(2 * N, 16)
    x1 = oc

    ip = _pad(arma_init, ((0, 0), (0, 0), (0, 14)))
    w0p = _pad(arma_W[0], ((0, 0), (0, 14), (0, 14)))
    r0p = _pad(arma_root[0], ((0, 0), (0, 0), (0, 14)))
    r1p = _pad(arma_root[1], ((0, 0), (0, 0), (0, 14)))
    b0p = _pad(arma_b[0].reshape(2, 50), ((0, 0), (0, 14)))
    b1p = _pad(arma_b[1].reshape(2, 50), ((0, 0), (0, 14)))

    z8 = _tc(
        _apre_body,
        jax.ShapeDtypeStruct((8, N, 16), jnp.float32),
        [bspec2, _rep((2, 32, 64))],
        bspec8, g,
    )(h2, ip).reshape(8 * N, 16)
    P8 = jnp.concatenate(
        [_prop_call(z8[2 * N * p:2 * N * (p + 1)], src2, dst2, norma, zer)
         for p in range(4)], axis=0)
    z8b = _tc(
        _amid_body,
        jax.ShapeDtypeStruct((8, N, 16), jnp.float32),
        [bspec8, bspec2, _rep((2, 32, 64)), _rep((2, 64)), _rep((2, 64, 64))],
        bspec8, g,
    )(P8.reshape(8, N, 16), h2, r0p, b0p, w0p).reshape(8 * N, 16)
    P8b = jnp.concatenate(
        [_prop_call(z8b[2 * N * p:2 * N * (p + 1)], src2, dst2, norma, zer)
         for p in range(4)], axis=0)
    out = _tc(
        _afin_body,
        jax.ShapeDtypeStruct((N, 2), jnp.float32),
        [bspec8, bspec2, _rep((2, 32, 64)), _rep((2, 64)),
         pl.BlockSpec((_BLK, 50), lambda i: (i, 0)), _rep((100, 2)), _rep((1, 2))],
        pl.BlockSpec((_BLK, 2), lambda i: (i, 0)), g,
    )(P8b.reshape(8, N, 16), h2, r1p, b1p, x1, W_out, b_out.reshape(1, 2))
    return out
